# baseline (speedup 1.0000x reference)
import functools

import jax
import jax.numpy as jnp
from jax import lax
from jax.experimental import pallas as pl
from jax.experimental.pallas import tpu as pltpu
from jax.experimental.pallas import tpu_sc as plsc

BATCH = 16384
BIT = 64
HALF = BIT // 2
L = 16
NS = 16
NTRAIN = 100000
PACK_BLK = 12288
PACK_GRID = 9
BPW = BATCH // NS
UCHUNK = 256
NCHUNK = BPW // UCHUNK


def _pack_body(ht_ref, p_ref):
    x = ht_ref[...]
    j = lax.broadcasted_iota(jnp.int32, (2, HALF), 1)
    q = lax.broadcasted_iota(jnp.int32, (2, HALF), 0)
    inq = (j >= q * 16) & (j < (q + 1) * 16)
    w = jnp.where(inq, jnp.left_shift(jnp.int32(1), j % 16), 0).astype(
        jnp.float32
    )
    dot = lax.dot_general(
        w, x, (((1,), (0,)), ((), ())), preferred_element_type=jnp.float32
    )
    p16 = ((jnp.float32(65535.0) - dot) * jnp.float32(0.5)).astype(jnp.int32)
    p_ref[...] = p16[0, :] + jnp.left_shift(p16[1, :], 16)


def _make_pack(g):
    return pl.pallas_call(
        _pack_body,
        grid=(PACK_GRID,),
        in_specs=[pl.BlockSpec((HALF, PACK_BLK), lambda i, g=g: (g, i))],
        out_specs=pl.BlockSpec((PACK_BLK,), lambda i: (i,)),
        out_shape=jax.ShapeDtypeStruct((NTRAIN,), jnp.int32),
        name=f"pack{g}",
    )


_pack = [_make_pack(0), _make_pack(1)]


def _make_mse(g):
    mesh = plsc.VectorSubcoreMesh(
        core_axis_name="c", subcore_axis_name="s", num_cores=1
    )
    jbase = g * HALF

    @functools.partial(
        pl.kernel,
        out_type=jax.ShapeDtypeStruct((NS, L), jnp.float32),
        mesh=mesh,
        scratch_types=[
            pltpu.VMEM((NTRAIN,), jnp.int32),
            pltpu.VMEM((BPW,), jnp.int32),
            pltpu.VMEM((2, HALF, UCHUNK), jnp.float32),
            pltpu.VMEM((L,), jnp.float32),
            pltpu.VMEM_SHARED((NTRAIN,), jnp.int32),
            pltpu.SemaphoreType.DMA,
            pltpu.SemaphoreType.DMA,
            pltpu.SemaphoreType.DMA,
        ],
        compiler_params=pltpu.CompilerParams(needs_layout_passes=False),
        name=f"mse{g}",
    )
    def _mse_bits(
        ut_hbm, ind_hbm, p_hbm, out_hbm, pg_v, ind_v, ut_v, acc_v, p_sh,
        psem, isem, usem,
    ):
        s = lax.axis_index("s")
        ibase = pl.multiple_of(s * BPW, BPW)

        pcopy = pltpu.make_async_copy(p_hbm, p_sh, psem)

        @pl.when(s == 0)
        def _():
            pcopy.start()

        icopy = pltpu.make_async_copy(
            ind_hbm.at[pl.ds(ibase, BPW)], ind_v, isem
        )
        icopy.start()

        def u_copy(c, buf):
            return pltpu.make_async_copy(
                ut_hbm.at[
                    pl.ds(jbase, HALF),
                    pl.ds(pl.multiple_of(ibase + c * UCHUNK, UCHUNK), UCHUNK),
                ],
                ut_v.at[buf],
                usem,
            )

        u_copy(0, 0).start()
        u_copy(1, 1).start()

        @pl.when(s == 0)
        def _():
            pcopy.wait()

        plsc.subcore_barrier()
        pltpu.sync_copy(p_sh, pg_v)
        icopy.wait()

        zero = jnp.zeros((L,), jnp.float32)
        accs = (zero, zero, zero, zero)
        for c in range(NCHUNK):
            buf = c % 2
            u_copy(c, buf).wait()
            cbase = c * UCHUNK

            @plsc.parallel_loop(0, UCHUNK, L, carry=accs)
            def body(k, a, cbase=cbase, buf=buf):
                idx = ind_v[pl.ds(cbase + k, L)]
                p = plsc.load_gather(pg_v, [idx])
                a = list(a)
                for jj in range(HALF):
                    neg = jnp.left_shift(p, 31 - jj) < 0
                    h = jnp.where(neg, jnp.float32(-1.0), jnp.float32(1.0))
                    d = ut_v[buf, jj, pl.ds(k, L)] - h
                    a[jj % 4] = a[jj % 4] + d * d
                return tuple(a)

            accs = body
            if c + 2 < NCHUNK:
                u_copy(c + 2, buf).start()
        acc_v[...] = (accs[0] + accs[1]) + (accs[2] + accs[3])
        pltpu.sync_copy(acc_v, out_hbm.at[s])

    return _mse_bits


_mse = [_make_mse(0), _make_mse(1)]


def kernel(u, y, ind, H):
    del y
    ht = H.T
    ut = u.T
    ind32 = ind.astype(jnp.int32)
    p0 = _pack[0](ht)
    p1 = _pack[1](ht)
    part0 = _mse[0](ut, ind32, p0)
    part1 = _mse[1](ut, ind32, p1)
    return (jnp.sum(part0) + jnp.sum(part1)) * (1.0 / (BATCH * BIT))


# speedup vs baseline: 1.2670x; 1.2670x over previous
import functools

import jax
import jax.numpy as jnp
from jax import lax
from jax.experimental import pallas as pl
from jax.experimental.pallas import tpu as pltpu
from jax.experimental.pallas import tpu_sc as plsc

BATCH = 16384
BIT = 64
HALF = BIT // 2
L = 16
NC = 2
NS = 16
NW = NC * NS
NTRAIN = 100000
PACK_BLK = 24576
PACK_GRID = 5
BPW = BATCH // NS
UCHUNK = 512
NCHUNK = BPW // UCHUNK


def _pack_body(ht_ref, p0_ref, p1_ref):
    x = ht_ref[...]
    j = lax.broadcasted_iota(jnp.int32, (4, BIT), 1)
    q = lax.broadcasted_iota(jnp.int32, (4, BIT), 0)
    inq = (j >= q * 16) & (j < (q + 1) * 16)
    w = jnp.where(inq, jnp.left_shift(jnp.int32(1), j % 16), 0).astype(
        jnp.float32
    )
    dot = lax.dot_general(
        w, x, (((1,), (0,)), ((), ())), preferred_element_type=jnp.float32
    )
    p16 = ((jnp.float32(65535.0) - dot) * jnp.float32(0.5)).astype(jnp.int32)
    p0_ref[...] = p16[0, :] + jnp.left_shift(p16[1, :], 16)
    p1_ref[...] = p16[2, :] + jnp.left_shift(p16[3, :], 16)


_pack = pl.pallas_call(
    _pack_body,
    grid=(PACK_GRID,),
    in_specs=[pl.BlockSpec((BIT, PACK_BLK), lambda i: (0, i))],
    out_specs=[
        pl.BlockSpec((PACK_BLK,), lambda i: (i,)),
        pl.BlockSpec((PACK_BLK,), lambda i: (i,)),
    ],
    out_shape=[
        jax.ShapeDtypeStruct((NTRAIN,), jnp.int32),
        jax.ShapeDtypeStruct((NTRAIN,), jnp.int32),
    ],
)

_mesh = plsc.VectorSubcoreMesh(core_axis_name="c", subcore_axis_name="s")


@functools.partial(
    pl.kernel,
    out_type=jax.ShapeDtypeStruct((NW, L), jnp.float32),
    mesh=_mesh,
    scratch_types=[
        pltpu.VMEM((NTRAIN,), jnp.int32),
        pltpu.VMEM((BPW,), jnp.int32),
        pltpu.VMEM((HALF, UCHUNK), jnp.float32),
        pltpu.VMEM((L,), jnp.float32),
        pltpu.VMEM_SHARED((NTRAIN,), jnp.int32),
        pltpu.SemaphoreType.DMA,
        pltpu.SemaphoreType.DMA,
        pltpu.SemaphoreType.DMA,
    ],
    compiler_params=pltpu.CompilerParams(needs_layout_passes=False),
)
def _mse_bits(
    ut_hbm, ind_hbm, p0_hbm, p1_hbm, out_hbm, pg_v, ind_v, ut_v, acc_v,
    p_sh, psem, isem, usem,
):
    g = lax.axis_index("c")
    s = lax.axis_index("s")
    wid = s * NC + g
    jbase = pl.multiple_of(g * HALF, HALF)
    ibase = pl.multiple_of(s * BPW, BPW)

    pcopy0 = pltpu.make_async_copy(p0_hbm, p_sh, psem)
    pcopy1 = pltpu.make_async_copy(p1_hbm, p_sh, psem)

    @pl.when(jnp.logical_and(s == 0, g == 0))
    def _():
        pcopy0.start()

    @pl.when(jnp.logical_and(s == 0, g != 0))
    def _():
        pcopy1.start()

    icopy = pltpu.make_async_copy(ind_hbm.at[pl.ds(ibase, BPW)], ind_v, isem)
    icopy.start()

    def u_copy(c):
        return pltpu.make_async_copy(
            ut_hbm.at[
                pl.ds(jbase, HALF),
                pl.ds(pl.multiple_of(ibase + c * UCHUNK, UCHUNK), UCHUNK),
            ],
            ut_v,
            usem,
        )

    u_copy(0).start()

    @pl.when(s == 0)
    def _():
        pcopy0.wait()

    plsc.subcore_barrier()
    pltpu.sync_copy(p_sh, pg_v)
    icopy.wait()

    zero = jnp.zeros((L,), jnp.float32)
    accs = (zero, zero, zero, zero)
    for c in range(NCHUNK):
        u_copy(c).wait()
        cbase = c * UCHUNK

        @plsc.parallel_loop(0, UCHUNK, L, carry=accs)
        def body(k, a, cbase=cbase):
            idx = ind_v[pl.ds(cbase + k, L)]
            p = plsc.load_gather(pg_v, [idx])
            a = list(a)
            for jj in range(HALF):
                neg = jnp.left_shift(p, 31 - jj) < 0
                h = jnp.where(neg, jnp.float32(-1.0), jnp.float32(1.0))
                d = ut_v[jj, pl.ds(k, L)] - h
                a[jj % 4] = a[jj % 4] + d * d
            return tuple(a)

        accs = body
        if c + 1 < NCHUNK:
            u_copy(c + 1).start()
    acc_v[...] = (accs[0] + accs[1]) + (accs[2] + accs[3])
    pltpu.sync_copy(acc_v, out_hbm.at[wid])


def kernel(u, y, ind, H):
    del y
    p0, p1 = _pack(H.T)
    partials = _mse_bits(u.T, ind.astype(jnp.int32), p0, p1)
    return jnp.sum(partials) * (1.0 / (BATCH * BIT))


# speedup vs baseline: 1.4244x; 1.1242x over previous
import functools

import jax
import jax.numpy as jnp
from jax import lax
from jax.experimental import pallas as pl
from jax.experimental.pallas import tpu as pltpu
from jax.experimental.pallas import tpu_sc as plsc

BATCH = 16384
BIT = 64
L = 16
NC = 2
NS = 16
NW = NC * NS
NTRAIN = 100000
TSPLIT = 51200
PACK_BLK = 10240
PACK_GRID = 5
NGRP = 4
GCOL = BIT // NGRP
BPW = BATCH // (NW // NGRP)


def _pack_body(lo_ref, hi_ref, t0_ref, t1_ref, t2_ref, t3_ref):
    j = lax.broadcasted_iota(jnp.int32, (NGRP, BIT), 1)
    q = lax.broadcasted_iota(jnp.int32, (NGRP, BIT), 0)
    inq = (j >= q * GCOL) & (j < (q + 1) * GCOL)
    w = jnp.where(inq, jnp.left_shift(jnp.int32(1), j % GCOL), 0).astype(
        jnp.float32
    )
    dn = (((1,), (0,)), ((), ()))
    dlo = lax.dot_general(w, lo_ref[...], dn, preferred_element_type=jnp.float32)
    dhi = lax.dot_general(w, hi_ref[...], dn, preferred_element_type=jnp.float32)
    plo = ((jnp.float32(65535.0) - dlo) * jnp.float32(0.5)).astype(jnp.int32)
    phi = ((jnp.float32(65535.0) - dhi) * jnp.float32(0.5)).astype(jnp.int32)
    t0_ref[...] = plo[0, :] + jnp.left_shift(phi[0, :], 16)
    t1_ref[...] = plo[1, :] + jnp.left_shift(phi[1, :], 16)
    t2_ref[...] = plo[2, :] + jnp.left_shift(phi[2, :], 16)
    t3_ref[...] = plo[3, :] + jnp.left_shift(phi[3, :], 16)


_pack = pl.pallas_call(
    _pack_body,
    grid=(PACK_GRID,),
    in_specs=[
        pl.BlockSpec((BIT, PACK_BLK), lambda i: (0, i)),
        pl.BlockSpec((BIT, PACK_BLK), lambda i: (0, i + PACK_GRID)),
    ],
    out_specs=[pl.BlockSpec((PACK_BLK,), lambda i: (i,)) for _ in range(NGRP)],
    out_shape=[
        jax.ShapeDtypeStruct((TSPLIT,), jnp.int32) for _ in range(NGRP)
    ],
)

_mesh = plsc.VectorSubcoreMesh(core_axis_name="c", subcore_axis_name="s")


@functools.partial(
    pl.kernel,
    out_type=jax.ShapeDtypeStruct((NW, L), jnp.float32),
    mesh=_mesh,
    scratch_types=[
        pltpu.VMEM((TSPLIT,), jnp.int32),
        pltpu.VMEM((BPW,), jnp.int32),
        pltpu.VMEM((GCOL, BPW), jnp.float32),
        pltpu.VMEM((L,), jnp.float32),
        pltpu.VMEM_SHARED((2, TSPLIT), jnp.int32),
        pltpu.SemaphoreType.DMA,
        pltpu.SemaphoreType.DMA,
        pltpu.SemaphoreType.DMA,
    ],
    compiler_params=pltpu.CompilerParams(needs_layout_passes=False),
)
def _mse_bits(
    ut_hbm, ind_hbm, t0_hbm, t1_hbm, t2_hbm, t3_hbm, out_hbm,
    pg_v, ind_v, ut_v, acc_v, p_sh, psem, isem, usem,
):
    c = lax.axis_index("c")
    s = lax.axis_index("s")
    half = s % 2
    b = s // 2
    wid = s * NC + c
    jbase = pl.multiple_of((c * 2 + half) * GCOL, GCOL)
    ibase = pl.multiple_of(b * BPW, BPW)

    for cc, hh, t_hbm in ((0, 0, t0_hbm), (0, 1, t1_hbm),
                          (1, 0, t2_hbm), (1, 1, t3_hbm)):
        @pl.when(jnp.logical_and(c == cc, jnp.logical_and(s == hh, True)))
        def _(t_hbm=t_hbm, hh=hh):
            pltpu.make_async_copy(t_hbm, p_sh.at[hh], psem).start()

    icopy = pltpu.make_async_copy(ind_hbm.at[pl.ds(ibase, BPW)], ind_v, isem)
    icopy.start()
    ucopy = pltpu.make_async_copy(
        ut_hbm.at[pl.ds(jbase, GCOL), pl.ds(ibase, BPW)], ut_v, usem
    )
    ucopy.start()

    @pl.when(s < 2)
    def _():
        pltpu.make_async_copy(t0_hbm, p_sh.at[0], psem).wait()

    plsc.subcore_barrier()
    pltpu.sync_copy(p_sh.at[half], pg_v)
    icopy.wait()
    ucopy.wait()

    zero = jnp.zeros((L,), jnp.float32)
    accs = (zero, zero, zero, zero)

    @plsc.parallel_loop(0, BPW, L, carry=accs)
    def body(k, a):
        idx = ind_v[pl.ds(k, L)]
        inlo = idx < TSPLIT
        widx = jnp.where(inlo, idx, idx - TSPLIT)
        p = plsc.load_gather(pg_v, [widx])
        shb = jnp.where(inlo, 31, 15)
        a = list(a)
        for jj in range(GCOL):
            neg = jnp.left_shift(p, shb - jj) < 0
            h = jnp.where(neg, jnp.float32(-1.0), jnp.float32(1.0))
            d = ut_v[jj, pl.ds(k, L)] - h
            a[jj % 4] = a[jj % 4] + d * d
        return tuple(a)

    accs = body
    acc_v[...] = (accs[0] + accs[1]) + (accs[2] + accs[3])
    pltpu.sync_copy(acc_v, out_hbm.at[wid])


def kernel(u, y, ind, H):
    del y
    t0, t1, t2, t3 = _pack(H.T, H.T)
    partials = _mse_bits(u.T, ind.astype(jnp.int32), t0, t1, t2, t3)
    return jnp.sum(partials) * (1.0 / (BATCH * BIT))


# speedup vs baseline: 1.4363x; 1.0084x over previous
import functools

import jax
import jax.numpy as jnp
from jax import lax
from jax.experimental import pallas as pl
from jax.experimental.pallas import tpu as pltpu
from jax.experimental.pallas import tpu_sc as plsc

BATCH = 16384
BIT = 64
L = 16
NC = 2
NS = 16
NW = NC * NS
NTRAIN = 100000
TSPLIT = 51200
PACK_BLK = 10240
PACK_GRID = 5
NGRP = 4
GCOL = BIT // NGRP
BPW = BATCH // (NW // NGRP)


def _pack_body(lo_ref, hi_ref, t0_ref, t1_ref, t2_ref, t3_ref):
    j = lax.broadcasted_iota(jnp.int32, (NGRP, BIT), 1)
    q = lax.broadcasted_iota(jnp.int32, (NGRP, BIT), 0)
    inq = (j >= q * GCOL) & (j < (q + 1) * GCOL)
    w = jnp.where(inq, jnp.left_shift(jnp.int32(1), j % GCOL), 0).astype(
        jnp.float32
    )
    dn = (((1,), (0,)), ((), ()))
    dlo = lax.dot_general(w, lo_ref[...], dn, preferred_element_type=jnp.float32)
    dhi = lax.dot_general(w, hi_ref[...], dn, preferred_element_type=jnp.float32)
    plo = ((jnp.float32(65535.0) - dlo) * jnp.float32(0.5)).astype(jnp.int32)
    phi = ((jnp.float32(65535.0) - dhi) * jnp.float32(0.5)).astype(jnp.int32)
    t0_ref[...] = plo[0, :] + jnp.left_shift(phi[0, :], 16)
    t1_ref[...] = plo[1, :] + jnp.left_shift(phi[1, :], 16)
    t2_ref[...] = plo[2, :] + jnp.left_shift(phi[2, :], 16)
    t3_ref[...] = plo[3, :] + jnp.left_shift(phi[3, :], 16)


_pack = pl.pallas_call(
    _pack_body,
    grid=(PACK_GRID,),
    in_specs=[
        pl.BlockSpec((BIT, PACK_BLK), lambda i: (0, i)),
        pl.BlockSpec((BIT, PACK_BLK), lambda i: (0, i + PACK_GRID)),
    ],
    out_specs=[pl.BlockSpec((PACK_BLK,), lambda i: (i,)) for _ in range(NGRP)],
    out_shape=[
        jax.ShapeDtypeStruct((TSPLIT,), jnp.int32) for _ in range(NGRP)
    ],
)

_mesh = plsc.VectorSubcoreMesh(core_axis_name="c", subcore_axis_name="s")


@functools.partial(
    pl.kernel,
    out_type=jax.ShapeDtypeStruct((NW, L), jnp.float32),
    mesh=_mesh,
    scratch_types=[
        pltpu.VMEM((TSPLIT,), jnp.int32),
        pltpu.VMEM((BPW,), jnp.int32),
        pltpu.VMEM((GCOL, BPW), jnp.float32),
        pltpu.VMEM((L,), jnp.float32),
        pltpu.VMEM_SHARED((2, TSPLIT), jnp.int32),
        pltpu.SemaphoreType.DMA,
        pltpu.SemaphoreType.DMA,
        pltpu.SemaphoreType.DMA,
    ],
    compiler_params=pltpu.CompilerParams(needs_layout_passes=False),
)
def _mse_bits(
    ut_hbm, ind_hbm, t0_hbm, t1_hbm, t2_hbm, t3_hbm, out_hbm,
    pg_v, ind_v, ut_v, acc_v, p_sh, psem, isem, usem,
):
    c = lax.axis_index("c")
    s = lax.axis_index("s")
    half = s % 2
    b = s // 2
    wid = s * NC + c
    jbase = pl.multiple_of((c * 2 + half) * GCOL, GCOL)
    ibase = pl.multiple_of(b * BPW, BPW)

    for cc, hh, t_hbm in ((0, 0, t0_hbm), (0, 1, t1_hbm),
                          (1, 0, t2_hbm), (1, 1, t3_hbm)):
        @pl.when(jnp.logical_and(c == cc, jnp.logical_and(s == hh, True)))
        def _(t_hbm=t_hbm, hh=hh):
            pltpu.make_async_copy(t_hbm, p_sh.at[hh], psem).start()

    icopy = pltpu.make_async_copy(ind_hbm.at[pl.ds(ibase, BPW)], ind_v, isem)
    icopy.start()
    ucopy = pltpu.make_async_copy(
        ut_hbm.at[pl.ds(jbase, GCOL), pl.ds(ibase, BPW)], ut_v, usem
    )
    ucopy.start()

    @pl.when(s < 2)
    def _():
        pltpu.make_async_copy(t0_hbm, p_sh.at[0], psem).wait()

    plsc.subcore_barrier()
    pltpu.sync_copy(p_sh.at[half], pg_v)
    icopy.wait()
    ucopy.wait()

    zero = jnp.zeros((L,), jnp.float32)
    accs = (zero, zero, zero, zero)

    @plsc.parallel_loop(0, BPW, L, carry=accs)
    def body(k, a):
        idx = ind_v[pl.ds(k, L)]
        inlo = idx < TSPLIT
        widx = jnp.where(inlo, idx, idx - TSPLIT)
        p = plsc.load_gather(pg_v, [widx])
        pp = jnp.left_shift(p, jnp.where(inlo, 16, 0))
        a = list(a)
        for jj in range(GCOL):
            neg = jnp.left_shift(pp, 15 - jj) < 0
            h = jnp.where(neg, jnp.float32(-1.0), jnp.float32(1.0))
            d = ut_v[jj, pl.ds(k, L)] - h
            a[jj % 4] = a[jj % 4] + d * d
        return tuple(a)

    accs = body
    acc_v[...] = (accs[0] + accs[1]) + (accs[2] + accs[3])
    pltpu.sync_copy(acc_v, out_hbm.at[wid])


def kernel(u, y, ind, H):
    del y
    t0, t1, t2, t3 = _pack(H.T, H.T)
    partials = _mse_bits(u.T, ind.astype(jnp.int32), t0, t1, t2, t3)
    return jnp.sum(partials) * (1.0 / (BATCH * BIT))
